# baseline (speedup 1.0000x reference)
import functools
import math

import jax
import jax.numpy as jnp
from jax import lax
from jax.experimental import pallas as pl
from jax.experimental.pallas import tpu as pltpu
from jax.experimental.pallas import tpu_sc as plsc

_info = plsc.get_sparse_core_info()
_NC, _NS, _L = _info.num_cores, _info.num_subcores, _info.num_lanes
_NW = _NC * _NS


def _make_lookup(BSZ: int, H: int, V: int, D: int, scale: float):
  assert BSZ % (_NW * _L) == 0 and D % _L == 0 and H % 2 == 0
  NB = BSZ // _NW
  n_groups = NB // _L
  n_outer = H // 2
  mesh = plsc.VectorSubcoreMesh(core_axis_name="c", subcore_axis_name="s")

  @functools.partial(
      pl.kernel,
      mesh=mesh,
      out_type=jax.ShapeDtypeStruct((H, D, BSZ), jnp.float32),
      compiler_params=pltpu.CompilerParams(needs_layout_passes=False),
      scratch_types=[
          [pltpu.VMEM((NB,), jnp.int32)] * 2,
          [pltpu.VMEM((NB,), jnp.int32)] * 2,
          [pltpu.VMEM((NB, 2 * D), jnp.float32)] * 2,
          [pltpu.VMEM((D, NB), jnp.float32)] * 2,
          [pltpu.SemaphoreType.DMA] * 2,
          [pltpu.SemaphoreType.DMA] * 2,
      ],
  )
  def lookup_kernel(table_hbm, seqt_hbm, out_hbm, idx_v, half_v, gbuf, tbuf,
                    gsem, ssem):
    wid = lax.axis_index("s") * _NC + lax.axis_index("c")
    b0 = wid * NB

    def load_and_fire(h, b):
      pltpu.sync_copy(seqt_hbm.at[h, pl.ds(b0, NB)], idx_v[b])

      def halve(jg, c2):
        sl = pl.ds(jg * _L, _L)
        half_v[b][sl] = idx_v[b][sl] >> 1
        return c2

      lax.fori_loop(0, n_groups, halve, 0, unroll=n_groups)
      pltpu.async_copy(table_hbm.at[half_v[b]], gbuf[b], gsem[b])

    for b in (0, 1):
      load_and_fire(b, b)

    def outer_body(g, carry):
      for b in (0, 1):
        h = 2 * g + b
        pltpu.make_async_copy(table_hbm.at[half_v[b]], gbuf[b],
                              gsem[b]).wait()
        @pl.when(g > 0)
        def _():
          pltpu.make_async_copy(tbuf[b], out_hbm.at[0, :, pl.ds(b0, NB)],
                                ssem[b]).wait()

        def comp(jg, c2):
          j0 = jg * _L
          sl = pl.ds(j0, _L)
          jids = j0 + lax.iota(jnp.int32, _L)
          colbase = (idx_v[b][sl] & 1) * D
          for c in range(D):
            vals = plsc.load_gather(gbuf[b], [jids, colbase + c])
            tbuf[b][c, sl] = vals * scale
          return c2

        lax.fori_loop(0, n_groups, comp, 0)

        pltpu.async_copy(tbuf[b], out_hbm.at[h, :, pl.ds(b0, NB)], ssem[b])

        @pl.when(g < n_outer - 1)
        def _():
          load_and_fire(h + 2, b)

      return carry

    lax.fori_loop(0, n_outer, outer_body, 0)

    for b in (0, 1):
      pltpu.make_async_copy(tbuf[b], out_hbm.at[0, :, pl.ds(b0, NB)],
                            ssem[b]).wait()

  return lookup_kernel


def kernel(seq, table):
  bsz, hist = seq.shape
  V, D = table.shape
  table_pairs = table.reshape(V // 2, 2 * D)
  out3 = _make_lookup(bsz, hist, V, D, math.sqrt(D))(table_pairs, seq.T)
  return jnp.transpose(out3, (2, 0, 1))
